# baseline (speedup 1.0000x reference)
import jax
import jax.numpy as jnp
from jax import lax
from jax.experimental import pallas as pl
from jax.experimental.pallas import tpu as pltpu, tpu_sc as plsc

N = 10000
E = 320000
D = 128
H = 32
A = 20
NE = 40
B = 500
NC, NS = 2, 16
NW = NC * NS
EPT = E // NW
CH = 80
NCH = EPT // CH
NP = 10240
ROWS_PER_TILE = NP // NS


def _deg_body(dst_hbm, deg_out, dst_v, deg_v):
    cid = lax.axis_index("c")
    sid = lax.axis_index("s")
    wid = cid * NS + sid
    pltpu.sync_copy(dst_hbm.at[wid], dst_v)

    def zero_body(i, _):
        deg_v[pl.ds(i * 16, 16)] = jnp.zeros((16,), jnp.float32)
        return 0

    lax.fori_loop(0, N // 16, zero_body, 0)
    ones = jnp.ones((16,), jnp.float32)

    def body(j, _):
        idx = dst_v[pl.ds(j * 16, 16)]
        plsc.addupdate_scatter(deg_v, [idx], ones)
        return 0

    lax.fori_loop(0, EPT // 16, body, 0)
    pltpu.sync_copy(deg_v, deg_out.at[wid])


def _deg_call(dst2):
    mesh = plsc.VectorSubcoreMesh(
        core_axis_name="c", subcore_axis_name="s", num_cores=NC, num_subcores=NS
    )
    return pl.kernel(
        _deg_body,
        out_type=jax.ShapeDtypeStruct((NW, N), jnp.float32),
        mesh=mesh,
        scratch_types=[
            pltpu.VMEM((EPT,), jnp.int32),
            pltpu.VMEM((N,), jnp.float32),
        ],
        compiler_params=pltpu.CompilerParams(needs_layout_passes=False),
    )(dst2)


def _mm_body(state_ref, w_ref, xw_ref):
    xw_ref[...] = jnp.dot(state_ref[...], w_ref[...], preferred_element_type=jnp.float32)


def _mm_call(state, w_gcn):
    blk = 1000
    return pl.pallas_call(
        _mm_body,
        grid=(N // blk,),
        in_specs=[
            pl.BlockSpec((blk, D), lambda i: (i, 0)),
            pl.BlockSpec((D, D), lambda i: (0, 0)),
        ],
        out_specs=pl.BlockSpec((blk, D), lambda i: (i, 0)),
        out_shape=jax.ShapeDtypeStruct((N, D), jnp.float32),
    )(state, w_gcn)


def _proj_body(xw_ref, deg_ref, y_ref):
    deg = jnp.sum(deg_ref[...], axis=1) + 1.0
    dinv = lax.rsqrt(deg)
    y_ref[...] = xw_ref[...] * dinv[:, None]


def _proj_call(xw, deg_part):
    blk = 1000
    return pl.pallas_call(
        _proj_body,
        grid=(N // blk,),
        in_specs=[
            pl.BlockSpec((blk, D), lambda i: (i, 0)),
            pl.BlockSpec((blk, NW), lambda i: (i, 0)),
        ],
        out_specs=pl.BlockSpec((blk, D), lambda i: (i, 0)),
        out_shape=jax.ShapeDtypeStruct((N, D), jnp.float32),
    )(xw, deg_part)


def _scat_body(y_hbm, src_hbm, dst_hbm, acc_out,
               src_v, dst_v, rows_a, acc_sh, sem_g, sem_s):
    cid = lax.axis_index("c")
    sid = lax.axis_index("s")
    wid = cid * NS + sid
    pltpu.sync_copy(src_hbm.at[wid], src_v)
    pltpu.sync_copy(dst_hbm.at[wid], dst_v)

    def zb(i, _):
        for t in range(D // 16):
            rows_a[i, pl.ds(t * 16, 16)] = jnp.zeros((16,), jnp.float32)
        return 0

    lax.fori_loop(0, CH, zb, 0)
    for k in range(ROWS_PER_TILE // CH):
        off = pl.multiple_of(sid * ROWS_PER_TILE + k * CH, 8)
        pltpu.sync_copy(rows_a, acc_sh.at[pl.ds(off, CH)])
    rem = ROWS_PER_TILE - (ROWS_PER_TILE // CH) * CH
    if rem:
        off = pl.multiple_of(sid * ROWS_PER_TILE + (ROWS_PER_TILE // CH) * CH, 8)
        pltpu.sync_copy(rows_a.at[pl.ds(0, rem)], acc_sh.at[pl.ds(off, rem)])
    plsc.subcore_barrier()

    def body(j, _):
        pltpu.async_copy(y_hbm.at[src_v.at[j]], rows_a, sem_g).wait()
        pltpu.async_copy(rows_a, acc_sh.at[dst_v.at[j]], sem_s, add=True).wait()
        return 0

    lax.fori_loop(0, NCH, body, 0)
    plsc.subcore_barrier()
    off = pl.multiple_of(sid * ROWS_PER_TILE, 8)
    pltpu.sync_copy(
        acc_sh.at[pl.ds(off, ROWS_PER_TILE)],
        acc_out.at[cid, pl.ds(off, ROWS_PER_TILE)],
    )


def _scat_call(y, src3, dst3):
    mesh = plsc.VectorSubcoreMesh(
        core_axis_name="c", subcore_axis_name="s", num_cores=NC, num_subcores=NS
    )
    return pl.kernel(
        _scat_body,
        out_type=jax.ShapeDtypeStruct((NC, NP, D), jnp.float32),
        mesh=mesh,
        scratch_types=[
            pltpu.VMEM((NCH, CH), jnp.int32),
            pltpu.VMEM((NCH, CH), jnp.int32),
            pltpu.VMEM((CH, D), jnp.float32),
            pltpu.VMEM_SHARED((NP, D), jnp.float32),
            pltpu.SemaphoreType.DMA,
            pltpu.SemaphoreType.DMA,
        ],
    )(y, src3, dst3)


def _epi_body(acc_ref, y_ref, state_ref, deg_ref, bg_ref, w1_ref, u_ref, v_ref):
    deg = jnp.sum(deg_ref[...], axis=1) + 1.0
    dinv = lax.rsqrt(deg)
    s = (acc_ref[0] + acc_ref[1] + y_ref[...]) * dinv[:, None] + bg_ref[...]
    z = jnp.maximum(s, 0.0) + state_ref[...]
    u_ref[...] = jnp.dot(z, w1_ref[0:D], preferred_element_type=jnp.float32)
    v_ref[...] = jnp.dot(z, w1_ref[D : 2 * D], preferred_element_type=jnp.float32)


def _epi_call(acc, y, state, deg_part, bg2, w_l1):
    blk = 1000
    return pl.pallas_call(
        _epi_body,
        grid=(N // blk,),
        in_specs=[
            pl.BlockSpec((NC, blk, D), lambda i: (0, i, 0)),
            pl.BlockSpec((blk, D), lambda i: (i, 0)),
            pl.BlockSpec((blk, D), lambda i: (i, 0)),
            pl.BlockSpec((blk, NW), lambda i: (i, 0)),
            pl.BlockSpec((1, D), lambda i: (0, 0)),
            pl.BlockSpec((2 * D, H), lambda i: (0, 0)),
        ],
        out_specs=[
            pl.BlockSpec((blk, H), lambda i: (i, 0)),
            pl.BlockSpec((blk, H), lambda i: (i, 0)),
        ],
        out_shape=[
            jax.ShapeDtypeStruct((N, H), jnp.float32),
            jax.ShapeDtypeStruct((N, H), jnp.float32),
        ],
    )(acc, y, state, deg_part, bg2, w_l1)


def _head_body(u_ref, v_ref, edges_ref, b1_ref, w2_ref, b2_ref, wmu_ref, bmu_ref, m_ref):
    es = edges_ref[:, 0].astype(jnp.float32)
    ed = edges_ref[:, 1].astype(jnp.float32)
    W = NE * H
    P = A * H
    rep = (
        jax.lax.broadcasted_iota(jnp.int32, (NE, W), 1) // H
        == jax.lax.broadcasted_iota(jnp.int32, (NE, W), 0)
    ).astype(jnp.float32)
    es_q = jnp.dot(es[None, :], rep, preferred_element_type=jnp.float32)
    ed_q = jnp.dot(ed[None, :], rep, preferred_element_type=jnp.float32)
    pi = jax.lax.broadcasted_iota(jnp.int32, (P, W), 0)
    qi = jax.lax.broadcasted_iota(jnp.int32, (P, W), 1)
    lane = (pi % H) == (qi % H)
    krow = (pi // H).astype(jnp.float32)
    one = jnp.float32(1.0)
    zero = jnp.float32(0.0)
    Ss = jnp.where(lane & (es_q == krow), one, zero)
    Sd = jnp.where(lane & (ed_q == krow), one, zero)
    pre = jnp.dot(u_ref[...], Ss, preferred_element_type=jnp.float32) + jnp.dot(
        v_ref[...], Sd, preferred_element_type=jnp.float32
    )
    h1 = jnp.concatenate([pre[:, a * H : (a + 1) * H] for a in range(NE)], axis=0)
    h1 = h1 + b1_ref[...]
    h1 = jnp.where(h1 > 0, h1, 0.01 * h1)
    h2 = jnp.dot(h1, w2_ref[...], preferred_element_type=jnp.float32) + b2_ref[...]
    h2 = jnp.where(h2 > 0, h2, 0.01 * h2)
    m = jnp.dot(h2, wmu_ref[...], preferred_element_type=jnp.float32) + bmu_ref[...] + 1e-10
    m_ref[...] = jnp.maximum(m, 0.0) + jnp.log1p(jnp.exp(-jnp.abs(m)))


def _head_call(u2, v2, edges, b1, w_l2, b2, w_mu, bmu):
    return pl.pallas_call(
        _head_body,
        out_shape=jax.ShapeDtypeStruct((B * NE, 1), jnp.float32),
    )(u2, v2, edges, b1, w_l2, b2, w_mu, bmu)


def kernel(state, edge_index, edges, W_gcn, b_gcn, W_l1, b_l1, W_l2, b_l2,
           W_mu, b_mu, W_sig, b_sig):
    src = edge_index[0]
    dst = edge_index[1]
    dst2 = dst.reshape(NW, EPT)
    src3 = src.reshape(NW, NCH, CH)
    dst3 = dst.reshape(NW, NCH, CH)

    xw = _mm_call(state, W_gcn)
    deg_part = _deg_call(dst2).T
    y = _proj_call(xw, deg_part)
    acc = _scat_call(y, src3, dst3)
    u, v = _epi_call(acc, y, state, deg_part, b_gcn.reshape(1, D), W_l1)
    m = _head_call(
        u.reshape(B, A * H),
        v.reshape(B, A * H),
        edges,
        b_l1.reshape(1, H),
        W_l2,
        b_l2.reshape(1, H),
        W_mu,
        b_mu.reshape(1, 1),
    )
    return m.reshape(NE, B).T


# speedup vs baseline: 1.0149x; 1.0149x over previous
import jax
import jax.numpy as jnp
from jax import lax
from jax.experimental import pallas as pl
from jax.experimental.pallas import tpu as pltpu, tpu_sc as plsc

N = 10000
E = 320000
D = 128
H = 32
A = 20
NE = 40
B = 500
NC, NS = 2, 16
NW = NC * NS
EPT = E // NW
CH = 80
NCH = EPT // CH
NP = 10240
ROWS_PER_TILE = NP // NS


def _deg_body(dst_hbm, deg_out, dst_v, deg_v):
    cid = lax.axis_index("c")
    sid = lax.axis_index("s")
    wid = cid * NS + sid
    pltpu.sync_copy(dst_hbm.at[wid], dst_v)

    def zero_body(i, _):
        deg_v[pl.ds(i * 16, 16)] = jnp.zeros((16,), jnp.float32)
        return 0

    lax.fori_loop(0, N // 16, zero_body, 0)
    ones = jnp.ones((16,), jnp.float32)

    def body(j, _):
        idx = dst_v[pl.ds(j * 16, 16)]
        plsc.addupdate_scatter(deg_v, [idx], ones)
        return 0

    lax.fori_loop(0, EPT // 16, body, 0)
    pltpu.sync_copy(deg_v, deg_out.at[wid])


def _deg_call(dst2):
    mesh = plsc.VectorSubcoreMesh(
        core_axis_name="c", subcore_axis_name="s", num_cores=NC, num_subcores=NS
    )
    return pl.kernel(
        _deg_body,
        out_type=jax.ShapeDtypeStruct((NW, N), jnp.float32),
        mesh=mesh,
        scratch_types=[
            pltpu.VMEM((EPT,), jnp.int32),
            pltpu.VMEM((N,), jnp.float32),
        ],
        compiler_params=pltpu.CompilerParams(needs_layout_passes=False),
    )(dst2)


def _proj_body(state_ref, w_ref, deg_ref, y_ref):
    deg = jnp.sum(deg_ref[...], axis=1) + 1.0
    dinv = lax.rsqrt(deg)
    xw = jnp.dot(state_ref[...], w_ref[...], preferred_element_type=jnp.float32)
    y_ref[...] = xw * dinv[:, None]


def _proj_call(state, w_gcn, deg_part):
    blk = 1000
    return pl.pallas_call(
        _proj_body,
        grid=(N // blk,),
        in_specs=[
            pl.BlockSpec((blk, D), lambda i: (i, 0)),
            pl.BlockSpec((D, D), lambda i: (0, 0)),
            pl.BlockSpec((blk, NW), lambda i: (i, 0)),
        ],
        out_specs=pl.BlockSpec((blk, D), lambda i: (i, 0)),
        out_shape=jax.ShapeDtypeStruct((N, D), jnp.float32),
    )(state, w_gcn, deg_part)


def _scat_body(y_hbm, src_hbm, dst_hbm, acc_out,
               src_v, dst_v, rows_a, acc_sh, sem_g, sem_s):
    cid = lax.axis_index("c")
    sid = lax.axis_index("s")
    wid = cid * NS + sid
    pltpu.sync_copy(src_hbm.at[wid], src_v)
    pltpu.sync_copy(dst_hbm.at[wid], dst_v)

    def zb(i, _):
        for t in range(D // 16):
            rows_a[i, pl.ds(t * 16, 16)] = jnp.zeros((16,), jnp.float32)
        return 0

    lax.fori_loop(0, CH, zb, 0)
    for k in range(ROWS_PER_TILE // CH):
        off = pl.multiple_of(sid * ROWS_PER_TILE + k * CH, 8)
        pltpu.sync_copy(rows_a, acc_sh.at[pl.ds(off, CH)])
    rem = ROWS_PER_TILE - (ROWS_PER_TILE // CH) * CH
    if rem:
        off = pl.multiple_of(sid * ROWS_PER_TILE + (ROWS_PER_TILE // CH) * CH, 8)
        pltpu.sync_copy(rows_a.at[pl.ds(0, rem)], acc_sh.at[pl.ds(off, rem)])
    plsc.subcore_barrier()

    def body(j, _):
        pltpu.async_copy(y_hbm.at[src_v.at[j]], rows_a, sem_g).wait()
        pltpu.async_copy(rows_a, acc_sh.at[dst_v.at[j]], sem_s, add=True).wait()
        return 0

    lax.fori_loop(0, NCH, body, 0)
    plsc.subcore_barrier()
    off = pl.multiple_of(sid * ROWS_PER_TILE, 8)
    pltpu.sync_copy(
        acc_sh.at[pl.ds(off, ROWS_PER_TILE)],
        acc_out.at[cid, pl.ds(off, ROWS_PER_TILE)],
    )


def _scat_call(y, src3, dst3):
    mesh = plsc.VectorSubcoreMesh(
        core_axis_name="c", subcore_axis_name="s", num_cores=NC, num_subcores=NS
    )
    return pl.kernel(
        _scat_body,
        out_type=jax.ShapeDtypeStruct((NC, NP, D), jnp.float32),
        mesh=mesh,
        scratch_types=[
            pltpu.VMEM((NCH, CH), jnp.int32),
            pltpu.VMEM((NCH, CH), jnp.int32),
            pltpu.VMEM((CH, D), jnp.float32),
            pltpu.VMEM_SHARED((NP, D), jnp.float32),
            pltpu.SemaphoreType.DMA,
            pltpu.SemaphoreType.DMA,
        ],
    )(y, src3, dst3)


def _epi_body(acc_ref, y_ref, state_ref, deg_ref, bg_ref, w1_ref, u_ref, v_ref):
    deg = jnp.sum(deg_ref[...], axis=1) + 1.0
    dinv = lax.rsqrt(deg)
    s = (acc_ref[0] + acc_ref[1] + y_ref[...]) * dinv[:, None] + bg_ref[...]
    z = jnp.maximum(s, 0.0) + state_ref[...]
    u_ref[...] = jnp.dot(z, w1_ref[0:D], preferred_element_type=jnp.float32)
    v_ref[...] = jnp.dot(z, w1_ref[D : 2 * D], preferred_element_type=jnp.float32)


def _epi_call(acc, y, state, deg_part, bg2, w_l1):
    blk = 1000
    return pl.pallas_call(
        _epi_body,
        grid=(N // blk,),
        in_specs=[
            pl.BlockSpec((NC, blk, D), lambda i: (0, i, 0)),
            pl.BlockSpec((blk, D), lambda i: (i, 0)),
            pl.BlockSpec((blk, D), lambda i: (i, 0)),
            pl.BlockSpec((blk, NW), lambda i: (i, 0)),
            pl.BlockSpec((1, D), lambda i: (0, 0)),
            pl.BlockSpec((2 * D, H), lambda i: (0, 0)),
        ],
        out_specs=[
            pl.BlockSpec((blk, H), lambda i: (i, 0)),
            pl.BlockSpec((blk, H), lambda i: (i, 0)),
        ],
        out_shape=[
            jax.ShapeDtypeStruct((N, H), jnp.float32),
            jax.ShapeDtypeStruct((N, H), jnp.float32),
        ],
    )(acc, y, state, deg_part, bg2, w_l1)


def _head_body(u_ref, v_ref, edges_ref, b1_ref, w2_ref, b2_ref, wmu_ref, bmu_ref, m_ref):
    es = edges_ref[:, 0].astype(jnp.float32)
    ed = edges_ref[:, 1].astype(jnp.float32)
    W = NE * H
    P = A * H
    rep = (
        jax.lax.broadcasted_iota(jnp.int32, (NE, W), 1) // H
        == jax.lax.broadcasted_iota(jnp.int32, (NE, W), 0)
    ).astype(jnp.float32)
    es_q = jnp.dot(es[None, :], rep, preferred_element_type=jnp.float32)
    ed_q = jnp.dot(ed[None, :], rep, preferred_element_type=jnp.float32)
    pi = jax.lax.broadcasted_iota(jnp.int32, (P, W), 0)
    qi = jax.lax.broadcasted_iota(jnp.int32, (P, W), 1)
    lane = (pi % H) == (qi % H)
    krow = (pi // H).astype(jnp.float32)
    one = jnp.float32(1.0)
    zero = jnp.float32(0.0)
    Ss = jnp.where(lane & (es_q == krow), one, zero)
    Sd = jnp.where(lane & (ed_q == krow), one, zero)
    pre = jnp.dot(u_ref[...], Ss, preferred_element_type=jnp.float32) + jnp.dot(
        v_ref[...], Sd, preferred_element_type=jnp.float32
    )
    h1 = jnp.concatenate([pre[:, a * H : (a + 1) * H] for a in range(NE)], axis=0)
    h1 = h1 + b1_ref[...]
    h1 = jnp.where(h1 > 0, h1, 0.01 * h1)
    h2 = jnp.dot(h1, w2_ref[...], preferred_element_type=jnp.float32) + b2_ref[...]
    h2 = jnp.where(h2 > 0, h2, 0.01 * h2)
    m = jnp.dot(h2, wmu_ref[...], preferred_element_type=jnp.float32) + bmu_ref[...] + 1e-10
    m_ref[...] = jnp.maximum(m, 0.0) + jnp.log1p(jnp.exp(-jnp.abs(m)))


def _head_call(u2, v2, edges, b1, w_l2, b2, w_mu, bmu):
    return pl.pallas_call(
        _head_body,
        out_shape=jax.ShapeDtypeStruct((B * NE, 1), jnp.float32),
    )(u2, v2, edges, b1, w_l2, b2, w_mu, bmu)


def kernel(state, edge_index, edges, W_gcn, b_gcn, W_l1, b_l1, W_l2, b_l2,
           W_mu, b_mu, W_sig, b_sig):
    src = edge_index[0]
    dst = edge_index[1]
    dst2 = dst.reshape(NW, EPT)
    src3 = src.reshape(NW, NCH, CH)
    dst3 = dst.reshape(NW, NCH, CH)

    deg_part = _deg_call(dst2).T
    y = _proj_call(state, W_gcn, deg_part)
    acc = _scat_call(y, src3, dst3)
    u, v = _epi_call(acc, y, state, deg_part, b_gcn.reshape(1, D), W_l1)
    m = _head_call(
        u.reshape(B, A * H),
        v.reshape(B, A * H),
        edges,
        b_l1.reshape(1, H),
        W_l2,
        b_l2.reshape(1, H),
        W_mu,
        b_mu.reshape(1, 1),
    )
    return m.reshape(NE, B).T
